# baseline (speedup 1.0000x reference)
import jax
import jax.numpy as jnp
from jax.experimental import pallas as pl

KEEP_RATIO = 0.5


def _scores_body(x_ref, o_ref):
    o_ref[...] = jnp.sum(jnp.abs(x_ref[...]), axis=-1)


def _mask_body(s_ref, o_ref):
    b, c = s_ref.shape
    k = max(1, min(c, int(round(c * KEEP_RATIO))))
    bits = jax.lax.bitcast_convert_type(s_ref[...], jnp.int32)
    col = jax.lax.broadcasted_iota(jnp.int32, (b, c), 1)

    def vsearch(_, carry):
        lo, hi = carry
        mid = lo + ((hi - lo + 1) >> 1)
        cnt = jnp.sum((bits >= mid).astype(jnp.int32), axis=1, keepdims=True)
        p = cnt >= k
        return jnp.where(p, mid, lo), jnp.where(p, hi, mid - 1)

    lo = jnp.zeros((b, 1), jnp.int32)
    hi = jnp.full((b, 1), 0x7F800000, jnp.int32)
    t, _ = jax.lax.fori_loop(0, 31, vsearch, (lo, hi))

    gt = bits > t
    eq = bits == t
    need_eq = k - jnp.sum(gt.astype(jnp.int32), axis=1, keepdims=True)

    def isearch(_, carry):
        lo2, hi2 = carry
        mid = (lo2 + hi2) >> 1
        cnt = jnp.sum((eq & (col <= mid)).astype(jnp.int32), axis=1,
                      keepdims=True)
        p = cnt >= need_eq
        return jnp.where(p, lo2, mid + 1), jnp.where(p, mid, hi2)

    lo2 = jnp.zeros((b, 1), jnp.int32)
    hi2 = jnp.full((b, 1), c - 1, jnp.int32)
    m, _ = jax.lax.fori_loop(0, 10, isearch, (lo2, hi2))

    o_ref[...] = (gt | (eq & (col <= m))).astype(jnp.float32)


def kernel(features, enabled):
    b, c, h, w = features.shape
    s = h * w
    x3 = features.reshape(b, c, s)

    bblk, cblk = 8, 128
    scores = pl.pallas_call(
        _scores_body,
        grid=(b // bblk, c // cblk),
        in_specs=[pl.BlockSpec((bblk, cblk, s), lambda i, j: (i, j, 0))],
        out_specs=pl.BlockSpec((bblk, cblk), lambda i, j: (i, j)),
        out_shape=jax.ShapeDtypeStruct((b, c), jnp.float32),
    )(x3)

    mask = pl.pallas_call(
        _mask_body,
        out_shape=jax.ShapeDtypeStruct((b, c), jnp.float32),
    )(scores)

    gated = mask.reshape(b, c, 1, 1)
    return jnp.where(jnp.asarray(enabled) != 0, gated,
                     jnp.ones_like(gated))
